# Initial kernel scaffold; baseline (speedup 1.0000x reference)
import jax
import jax.numpy as jnp
from jax.experimental import pallas as pl


def kernel(student_features, teacher_features):
    raise NotImplementedError("write your pallas kernel here")


# speedup vs baseline: 2.6504x; 2.6504x over previous
import functools

import jax
import jax.numpy as jnp
from jax import lax
from jax.experimental import pallas as pl
from jax.experimental.pallas import tpu as pltpu

B, S, D, E = 2, 2048, 1024, 8
S_BLK = 512
NS = S // S_BLK
EPS = 1e-12


def _sims_kernel(s_ref, t_ref, idx_ref, acc_ref):
    s = pl.program_id(0)
    e = pl.program_id(1)
    for b in range(B):
        sf = s_ref[b]
        tf = t_ref[0, b]
        dot = jnp.sum(sf * tf, axis=1, keepdims=True)
        tn2 = jnp.sum(tf * tf, axis=1, keepdims=True)
        sn2 = jnp.sum(sf * sf, axis=1, keepdims=True)
        denom = jnp.maximum(jnp.sqrt(sn2), EPS) * jnp.maximum(jnp.sqrt(tn2), EPS)
        part = jnp.sum(dot / denom)
        prev = jnp.where(s == 0, jnp.float32(0.0), acc_ref[b, e])
        acc_ref[b, e] = prev + part

    @pl.when((s == NS - 1) & (e == E - 1))
    def _():
        for b in range(B):
            def body(ei, c):
                bv, bi = c
                v = acc_ref[b, ei]
                take = v > bv
                return (jnp.where(take, v, bv), jnp.where(take, ei, bi))

            _, bi = lax.fori_loop(1, E, body, (acc_ref[b, 0], jnp.int32(0)))
            idx_ref[b] = bi.astype(jnp.int32)


def _copy_kernel(idx_ref, t_ref, o_ref):
    del idx_ref
    o_ref[...] = t_ref[0]


@functools.partial(jax.jit, static_argnames=("interpret",))
def kernel(student_features, teacher_features, interpret=False):
    idx = pl.pallas_call(
        _sims_kernel,
        grid=(NS, E),
        in_specs=[
            pl.BlockSpec((B, S_BLK, D), lambda s, e: (0, s, 0)),
            pl.BlockSpec((1, B, S_BLK, D), lambda s, e: (e, 0, s, 0)),
        ],
        out_specs=pl.BlockSpec(memory_space=pltpu.SMEM),
        out_shape=jax.ShapeDtypeStruct((B,), jnp.int32),
        scratch_shapes=[pltpu.SMEM((B, E), jnp.float32)],
        compiler_params=pltpu.CompilerParams(
            dimension_semantics=("arbitrary", "arbitrary"),
        ),
        interpret=interpret,
    )(student_features, teacher_features)

    grid_spec = pltpu.PrefetchScalarGridSpec(
        num_scalar_prefetch=1,
        grid=(B, NS),
        in_specs=[
            pl.BlockSpec((1, 1, S_BLK, D), lambda b, s, idx_ref: (idx_ref[b], b, s, 0)),
        ],
        out_specs=pl.BlockSpec((1, S_BLK, D), lambda b, s, idx_ref: (b, s, 0)),
    )
    out = pl.pallas_call(
        _copy_kernel,
        grid_spec=grid_spec,
        out_shape=jax.ShapeDtypeStruct((B, S, D), jnp.float32),
        interpret=interpret,
    )(idx, teacher_features)
    return out
